# Initial kernel scaffold; baseline (speedup 1.0000x reference)
import jax
import jax.numpy as jnp
from jax.experimental import pallas as pl


def kernel(input, edge_index, W, b):
    raise NotImplementedError("write your pallas kernel here")


# speedup vs baseline: 7.3781x; 7.3781x over previous
import functools

import jax
import jax.numpy as jnp
from jax import lax
from jax.experimental import pallas as pl
from jax.experimental.pallas import tpu as pltpu
from jax.experimental.pallas import tpu_sc as plsc

N_NODES = 10000
N_EDGES = 320000
F = 128

NC = 2
NS = 16
NW = NC * NS

EPW = N_EDGES // NW
CHUNK = 80
NCHUNK = EPW // CHUNK

N_PAD = 10240
ROWS_PER_TILE = N_PAD // NS
ZR = CHUNK
NZC = ROWS_PER_TILE // ZR


def _matmul_body(x_ref, w_ref, o_ref):
    o_ref[...] = jnp.dot(x_ref[...], w_ref[...],
                         preferred_element_type=jnp.float32)


def _support_matmul(x, w):
    grid = 10
    rows = N_NODES // grid
    return pl.pallas_call(
        _matmul_body,
        grid=(grid,),
        in_specs=[
            pl.BlockSpec((rows, F), lambda i: (i, 0)),
            pl.BlockSpec((F, F), lambda i: (0, 0)),
        ],
        out_specs=pl.BlockSpec((rows, F), lambda i: (i, 0)),
        out_shape=jax.ShapeDtypeStruct((N_NODES, F), jnp.float32),
    )(x, w)


def _aggregate_body(src_hbm, dst_hbm, support_hbm, zeros_hbm, part_hbm,
                    sidx_v, didx_v, rows_v, acc_sh, sem):
    c = lax.axis_index("c")
    s = lax.axis_index("s")
    wid = c * NS + s

    pltpu.sync_copy(zeros_hbm, rows_v)
    row0 = s * ROWS_PER_TILE
    for k in range(NZC):
        pltpu.sync_copy(rows_v, acc_sh.at[pl.ds(row0 + k * ZR, ZR)])
    plsc.subcore_barrier()

    pltpu.sync_copy(src_hbm.at[wid], sidx_v)
    pltpu.sync_copy(dst_hbm.at[wid], didx_v)

    def body(i, carry):
        pltpu.async_copy(support_hbm.at[sidx_v.at[i]], rows_v, sem).wait()
        pltpu.sync_copy(rows_v, acc_sh.at[didx_v.at[i]], add=True)
        return carry

    lax.fori_loop(0, NCHUNK, body, 0)
    plsc.subcore_barrier()

    for k in range(NZC):
        r = row0 + k * ZR
        pltpu.sync_copy(acc_sh.at[pl.ds(r, ZR)], rows_v)
        pltpu.sync_copy(rows_v, part_hbm.at[c, pl.ds(r, ZR)])


def _aggregate(src, dst, support, zeros):
    mesh = plsc.VectorSubcoreMesh(core_axis_name="c", subcore_axis_name="s")
    kern = functools.partial(
        pl.kernel,
        out_type=jax.ShapeDtypeStruct((NC, N_PAD, F), jnp.float32),
        mesh=mesh,
        scratch_types=[
            pltpu.VMEM((NCHUNK, CHUNK), jnp.int32),
            pltpu.VMEM((NCHUNK, CHUNK), jnp.int32),
            pltpu.VMEM((CHUNK, F), jnp.float32),
            pltpu.VMEM_SHARED((N_PAD, F), jnp.float32),
            pltpu.SemaphoreType.DMA,
        ],
    )(_aggregate_body)
    return kern(src, dst, support, zeros)


def _combine_body(p_ref, b_ref, o_ref):
    o_ref[...] = p_ref[0] + p_ref[1] + b_ref[...]


def _combine(part, b):
    grid = 10
    rows = N_NODES // grid
    return pl.pallas_call(
        _combine_body,
        grid=(grid,),
        in_specs=[
            pl.BlockSpec((NC, rows, F), lambda i: (0, i, 0)),
            pl.BlockSpec((1, F), lambda i: (0, 0)),
        ],
        out_specs=pl.BlockSpec((rows, F), lambda i: (i, 0)),
        out_shape=jax.ShapeDtypeStruct((N_NODES, F), jnp.float32),
    )(part, b.reshape(1, F))


def kernel(input, edge_index, W, b):
    x = input
    ei = edge_index.astype(jnp.int32)
    dst = ei[0].reshape(NW, NCHUNK, CHUNK)
    src = ei[1].reshape(NW, NCHUNK, CHUNK)
    support = _support_matmul(x, W)
    zeros = jnp.zeros((ZR, F), dtype=jnp.float32)
    part = _aggregate(src, dst, support, zeros)
    return _combine(part, b)


# speedup vs baseline: 11.3915x; 1.5440x over previous
import functools

import jax
import jax.numpy as jnp
from jax import lax
from jax.experimental import pallas as pl
from jax.experimental.pallas import tpu as pltpu
from jax.experimental.pallas import tpu_sc as plsc

N_NODES = 10000
N_EDGES = 320000
F = 128

NC = 2
NS = 16
NW = NC * NS

EPW = N_EDGES // NW
CHUNK = 80
NCHUNK = EPW // CHUNK

N_PAD = 10240
ROWS_PER_TILE = N_PAD // NS
ZR = CHUNK
NZC = ROWS_PER_TILE // ZR


def _matmul_body(x_ref, w_ref, o_ref):
    o_ref[...] = jnp.dot(x_ref[...], w_ref[...],
                         preferred_element_type=jnp.float32)


def _support_matmul(x, w):
    grid = 10
    rows = N_NODES // grid
    return pl.pallas_call(
        _matmul_body,
        grid=(grid,),
        in_specs=[
            pl.BlockSpec((rows, F), lambda i: (i, 0)),
            pl.BlockSpec((F, F), lambda i: (0, 0)),
        ],
        out_specs=pl.BlockSpec((rows, F), lambda i: (i, 0)),
        out_shape=jax.ShapeDtypeStruct((N_NODES, F), jnp.float32),
    )(x, w)


def _aggregate_body(src_hbm, dst_hbm, support_hbm, zeros_hbm, part_hbm,
                    sidx_v, didx_v, rows0_v, rows1_v, acc_sh, sem0, sem1):
    c = lax.axis_index("c")
    s = lax.axis_index("s")
    wid = c * NS + s
    rows = (rows0_v, rows1_v)
    sems = (sem0, sem1)

    pltpu.sync_copy(zeros_hbm, rows0_v)
    row0 = s * ROWS_PER_TILE
    for k in range(NZC):
        pltpu.sync_copy(rows0_v, acc_sh.at[pl.ds(row0 + k * ZR, ZR)])
    plsc.subcore_barrier()

    pltpu.sync_copy(src_hbm.at[wid], sidx_v)
    pltpu.sync_copy(dst_hbm.at[wid], didx_v)

    def gather_start(i, b):
        pltpu.async_copy(support_hbm.at[sidx_v.at[i]], rows[b], sems[b])

    def gather_wait(i, b):
        pltpu.make_async_copy(support_hbm.at[sidx_v.at[i]], rows[b],
                              sems[b]).wait()

    def scatter(i, b):
        pltpu.sync_copy(rows[b], acc_sh.at[didx_v.at[i]], add=True)

    gather_start(0, 0)

    def body(j, carry):
        i0 = 2 * j
        gather_start(i0 + 1, 1)
        gather_wait(i0, 0)
        scatter(i0, 0)
        gather_start(i0 + 2, 0)
        gather_wait(i0 + 1, 1)
        scatter(i0 + 1, 1)
        return carry

    lax.fori_loop(0, (NCHUNK - 1) // 2, body, 0)
    gather_wait(NCHUNK - 1, 0)
    scatter(NCHUNK - 1, 0)
    plsc.subcore_barrier()

    for k in range(NZC):
        r = row0 + k * ZR
        pltpu.sync_copy(acc_sh.at[pl.ds(r, ZR)], rows0_v)
        pltpu.sync_copy(rows0_v, part_hbm.at[c, pl.ds(r, ZR)])


def _aggregate(src, dst, support, zeros):
    mesh = plsc.VectorSubcoreMesh(core_axis_name="c", subcore_axis_name="s")
    kern = functools.partial(
        pl.kernel,
        out_type=jax.ShapeDtypeStruct((NC, N_PAD, F), jnp.float32),
        mesh=mesh,
        compiler_params=pltpu.CompilerParams(use_tc_tiling_on_sc=False),
        scratch_types=[
            pltpu.VMEM((NCHUNK, CHUNK), jnp.int32),
            pltpu.VMEM((NCHUNK, CHUNK), jnp.int32),
            pltpu.VMEM((CHUNK, F), jnp.float32),
            pltpu.VMEM((CHUNK, F), jnp.float32),
            pltpu.VMEM_SHARED((N_PAD, F), jnp.float32),
            pltpu.SemaphoreType.DMA,
            pltpu.SemaphoreType.DMA,
        ],
    )(_aggregate_body)
    return kern(src, dst, support, zeros)


def _combine_body(p_ref, b_ref, o_ref):
    o_ref[...] = p_ref[0] + p_ref[1] + b_ref[...]


def _combine(part, b):
    grid = 10
    rows = N_NODES // grid
    return pl.pallas_call(
        _combine_body,
        grid=(grid,),
        in_specs=[
            pl.BlockSpec((NC, rows, F), lambda i: (0, i, 0)),
            pl.BlockSpec((1, F), lambda i: (0, 0)),
        ],
        out_specs=pl.BlockSpec((rows, F), lambda i: (i, 0)),
        out_shape=jax.ShapeDtypeStruct((N_NODES, F), jnp.float32),
    )(part, b.reshape(1, F))


def kernel(input, edge_index, W, b):
    x = input
    ei = edge_index.astype(jnp.int32)
    dst = ei[0].reshape(NW, NCHUNK, CHUNK)
    src = ei[1].reshape(NW, NCHUNK, CHUNK)
    support = _support_matmul(x, W)
    zeros = jnp.zeros((ZR, F), dtype=jnp.float32)
    part = _aggregate(src, dst, support, zeros)
    return _combine(part, b)


# speedup vs baseline: 11.9686x; 1.0507x over previous
import functools

import jax
import jax.numpy as jnp
from jax import lax
from jax.experimental import pallas as pl
from jax.experimental.pallas import tpu as pltpu
from jax.experimental.pallas import tpu_sc as plsc

N_NODES = 10000
N_EDGES = 320000
F = 128

NC = 2
NS = 16
NW = NC * NS

EPW = N_EDGES // NW
CHUNK = 80
NCHUNK = EPW // CHUNK

N_PAD = 10240
ROWS_PER_TILE = N_PAD // NS
ZR = CHUNK
NZC = ROWS_PER_TILE // ZR


def _aggregate_body(src_hbm, dst_hbm, x_hbm, zeros_hbm, part_hbm,
                    sidx_v, didx_v, rows0_v, rows1_v, acc_sh, sem0, sem1):
    c = lax.axis_index("c")
    s = lax.axis_index("s")
    wid = c * NS + s
    rows = (rows0_v, rows1_v)
    sems = (sem0, sem1)

    pltpu.sync_copy(zeros_hbm, rows0_v)
    row0 = s * ROWS_PER_TILE
    for k in range(NZC):
        pltpu.sync_copy(rows0_v, acc_sh.at[pl.ds(row0 + k * ZR, ZR)])
    plsc.subcore_barrier()

    pltpu.sync_copy(src_hbm.at[wid], sidx_v)
    pltpu.sync_copy(dst_hbm.at[wid], didx_v)

    def gather_start(i, b):
        pltpu.async_copy(x_hbm.at[sidx_v.at[i]], rows[b], sems[b])

    def gather_wait(i, b):
        pltpu.make_async_copy(x_hbm.at[sidx_v.at[i]], rows[b],
                              sems[b]).wait()

    def scatter(i, b):
        pltpu.sync_copy(rows[b], acc_sh.at[didx_v.at[i]], add=True)

    gather_start(0, 0)

    def body(j, carry):
        i0 = 2 * j
        gather_start(i0 + 1, 1)
        gather_wait(i0, 0)
        scatter(i0, 0)
        gather_start(i0 + 2, 0)
        gather_wait(i0 + 1, 1)
        scatter(i0 + 1, 1)
        return carry

    lax.fori_loop(0, (NCHUNK - 1) // 2, body, 0)
    gather_wait(NCHUNK - 1, 0)
    scatter(NCHUNK - 1, 0)
    plsc.subcore_barrier()

    for k in range(NZC):
        r = row0 + k * ZR
        pltpu.sync_copy(acc_sh.at[pl.ds(r, ZR)], rows0_v)
        pltpu.sync_copy(rows0_v, part_hbm.at[c, pl.ds(r, ZR)])


def _aggregate(src, dst, x, zeros):
    mesh = plsc.VectorSubcoreMesh(core_axis_name="c", subcore_axis_name="s")
    kern = functools.partial(
        pl.kernel,
        out_type=jax.ShapeDtypeStruct((NC, N_PAD, F), jnp.float32),
        mesh=mesh,
        compiler_params=pltpu.CompilerParams(use_tc_tiling_on_sc=False),
        scratch_types=[
            pltpu.VMEM((NCHUNK, CHUNK), jnp.int32),
            pltpu.VMEM((NCHUNK, CHUNK), jnp.int32),
            pltpu.VMEM((CHUNK, F), jnp.float32),
            pltpu.VMEM((CHUNK, F), jnp.float32),
            pltpu.VMEM_SHARED((N_PAD, F), jnp.float32),
            pltpu.SemaphoreType.DMA,
            pltpu.SemaphoreType.DMA,
        ],
    )(_aggregate_body)
    return kern(src, dst, x, zeros)


def _combine_matmul_body(p_ref, w_ref, b_ref, o_ref):
    agg = p_ref[0] + p_ref[1]
    o_ref[...] = jnp.dot(agg, w_ref[...],
                         preferred_element_type=jnp.float32) + b_ref[...]


def _combine_matmul(part, w, b):
    grid = 10
    rows = N_NODES // grid
    return pl.pallas_call(
        _combine_matmul_body,
        grid=(grid,),
        in_specs=[
            pl.BlockSpec((NC, rows, F), lambda i: (0, i, 0)),
            pl.BlockSpec((F, F), lambda i: (0, 0)),
            pl.BlockSpec((1, F), lambda i: (0, 0)),
        ],
        out_specs=pl.BlockSpec((rows, F), lambda i: (i, 0)),
        out_shape=jax.ShapeDtypeStruct((N_NODES, F), jnp.float32),
    )(part, w, b.reshape(1, F))


def kernel(input, edge_index, W, b):
    x = input
    ei = edge_index.astype(jnp.int32)
    dst = ei[0].reshape(NW, NCHUNK, CHUNK)
    src = ei[1].reshape(NW, NCHUNK, CHUNK)
    zeros = jnp.zeros((ZR, F), dtype=jnp.float32)
    part = _aggregate(src, dst, x, zeros)
    return _combine_matmul(part, W, b)
